# baseline (device time: 115093 ns/iter reference)
import jax
import jax.numpy as jnp
from jax import lax
from jax.experimental import pallas as pl
from jax.experimental.pallas import tpu as pltpu

K = 4


def kernel(x):
    _, m, n2 = x.shape
    n = n2 // 2
    mc = m // K

    def body(
        x_hbm,
        out_ref,
        stage_ref,
        send_bf16,
        comm_bf16,
        send_sems,
        recv_sems,
        stage_sems,
        keep_sem,
    ):
        my_x = lax.axis_index("x")
        my_y = lax.axis_index("y")
        my_z = lax.axis_index("z")
        other_x = 1 - my_x

        barrier_sem = pltpu.get_barrier_semaphore()
        pl.semaphore_signal(
            barrier_sem, inc=1,
            device_id=(other_x, my_y, my_z),
            device_id_type=pl.DeviceIdType.MESH,
        )
        pl.semaphore_wait(barrier_sem, 1)

        def exchange(keep_lo, send_lo):
            keep_cp = pltpu.make_async_copy(
                x_hbm.at[0, :, pl.ds(keep_lo, n)], out_ref, keep_sem
            )
            keep_cp.start()

            stage_cps = []
            for k in range(K):
                rows = pl.ds(k * mc, mc)
                cp = pltpu.make_async_copy(
                    x_hbm.at[0, rows, pl.ds(send_lo, n)],
                    stage_ref.at[rows],
                    stage_sems.at[k],
                )
                cp.start()
                stage_cps.append(cp)

            rdmas = []
            for k in range(K):
                rows = pl.ds(k * mc, mc)
                stage_cps[k].wait()
                send_bf16[rows, :] = stage_ref[rows, :].astype(jnp.bfloat16)
                rdma = pltpu.make_async_remote_copy(
                    src_ref=send_bf16.at[rows],
                    dst_ref=comm_bf16.at[rows],
                    send_sem=send_sems.at[k],
                    recv_sem=recv_sems.at[k],
                    device_id=(other_x, my_y, my_z),
                    device_id_type=pl.DeviceIdType.MESH,
                )
                rdma.start()
                rdmas.append(rdma)

            keep_cp.wait()
            for k in range(K):
                rows = pl.ds(k * mc, mc)
                rdmas[k].wait_recv()
                out_ref[rows, :] = out_ref[rows, :] + comm_bf16[rows, :].astype(
                    jnp.float32
                )
            for k in range(K):
                rdmas[k].wait_send()

        @pl.when(my_x == 0)
        def _():
            exchange(0, n)

        @pl.when(my_x == 1)
        def _():
            exchange(n, 0)

    return pl.pallas_call(
        body,
        out_shape=jax.ShapeDtypeStruct((m, n), x.dtype),
        in_specs=[pl.BlockSpec(memory_space=pl.ANY)],
        out_specs=pl.BlockSpec(memory_space=pltpu.VMEM),
        scratch_shapes=[
            pltpu.VMEM((m, n), jnp.float32),
            pltpu.VMEM((m, n), jnp.bfloat16),
            pltpu.VMEM((m, n), jnp.bfloat16),
            pltpu.SemaphoreType.DMA((K,)),
            pltpu.SemaphoreType.DMA((K,)),
            pltpu.SemaphoreType.DMA((K,)),
            pltpu.SemaphoreType.DMA,
        ],
        compiler_params=pltpu.CompilerParams(
            collective_id=0,
            vmem_limit_bytes=60 * 1024 * 1024,
        ),
    )(x)


# device time: 109804 ns/iter; 1.0482x vs baseline; 1.0482x over previous
import jax
import jax.numpy as jnp
from jax import lax
from jax.experimental import pallas as pl
from jax.experimental.pallas import tpu as pltpu

K = 8


def kernel(x):
    _, m, n2 = x.shape
    n = n2 // 2
    mc = m // K

    def body(
        x_hbm,
        out_hbm,
        res_ref,
        stage_ref,
        send_bf16,
        comm_bf16,
        send_sems,
        recv_sems,
        stage_sems,
        out_sems,
        keep_sem,
    ):
        my_x = lax.axis_index("x")
        my_y = lax.axis_index("y")
        my_z = lax.axis_index("z")
        other_x = 1 - my_x

        barrier_sem = pltpu.get_barrier_semaphore()
        pl.semaphore_signal(
            barrier_sem, inc=1,
            device_id=(other_x, my_y, my_z),
            device_id_type=pl.DeviceIdType.MESH,
        )
        pl.semaphore_wait(barrier_sem, 1)

        def exchange(keep_lo, send_lo):
            keep_cp = pltpu.make_async_copy(
                x_hbm.at[0, :, pl.ds(keep_lo, n)], res_ref, keep_sem
            )
            keep_cp.start()

            stage_cps = []
            for k in range(K):
                rows = pl.ds(k * mc, mc)
                cp = pltpu.make_async_copy(
                    x_hbm.at[0, rows, pl.ds(send_lo, n)],
                    stage_ref.at[rows],
                    stage_sems.at[k],
                )
                cp.start()
                stage_cps.append(cp)

            rdmas = []
            for k in range(K):
                rows = pl.ds(k * mc, mc)
                stage_cps[k].wait()
                send_bf16[rows, :] = stage_ref[rows, :].astype(jnp.bfloat16)
                rdma = pltpu.make_async_remote_copy(
                    src_ref=send_bf16.at[rows],
                    dst_ref=comm_bf16.at[rows],
                    send_sem=send_sems.at[k],
                    recv_sem=recv_sems.at[k],
                    device_id=(other_x, my_y, my_z),
                    device_id_type=pl.DeviceIdType.MESH,
                )
                rdma.start()
                rdmas.append(rdma)

            keep_cp.wait()
            out_cps = []
            for k in range(K):
                rows = pl.ds(k * mc, mc)
                rdmas[k].wait_recv()
                res_ref[rows, :] = res_ref[rows, :] + comm_bf16[rows, :].astype(
                    jnp.float32
                )
                ocp = pltpu.make_async_copy(
                    res_ref.at[rows], out_hbm.at[rows], out_sems.at[k]
                )
                ocp.start()
                out_cps.append(ocp)
            for k in range(K):
                out_cps[k].wait()
                rdmas[k].wait_send()

        @pl.when(my_x == 0)
        def _():
            exchange(0, n)

        @pl.when(my_x == 1)
        def _():
            exchange(n, 0)

    return pl.pallas_call(
        body,
        out_shape=jax.ShapeDtypeStruct((m, n), x.dtype),
        in_specs=[pl.BlockSpec(memory_space=pl.ANY)],
        out_specs=pl.BlockSpec(memory_space=pl.ANY),
        scratch_shapes=[
            pltpu.VMEM((m, n), jnp.float32),
            pltpu.VMEM((m, n), jnp.float32),
            pltpu.VMEM((m, n), jnp.bfloat16),
            pltpu.VMEM((m, n), jnp.bfloat16),
            pltpu.SemaphoreType.DMA((K,)),
            pltpu.SemaphoreType.DMA((K,)),
            pltpu.SemaphoreType.DMA((K,)),
            pltpu.SemaphoreType.DMA((K,)),
            pltpu.SemaphoreType.DMA,
        ],
        compiler_params=pltpu.CompilerParams(
            collective_id=0,
            vmem_limit_bytes=60 * 1024 * 1024,
        ),
    )(x)


# device time: 109144 ns/iter; 1.0545x vs baseline; 1.0060x over previous
import jax
import jax.numpy as jnp
from jax import lax
from jax.experimental import pallas as pl
from jax.experimental.pallas import tpu as pltpu

CHUNK_ROWS = (128, 256, 512, 704, 704, 704, 704, 384)
K = len(CHUNK_ROWS)
CHUNK_OFF = tuple(sum(CHUNK_ROWS[:i]) for i in range(K))


def kernel(x):
    _, m, n2 = x.shape
    n = n2 // 2
    assert sum(CHUNK_ROWS) == m

    def body(
        x_hbm,
        out_hbm,
        res_ref,
        stage_ref,
        send_bf16,
        comm_bf16,
        send_sems,
        recv_sems,
        stage_sems,
        out_sems,
        keep_sem,
    ):
        my_x = lax.axis_index("x")
        my_y = lax.axis_index("y")
        my_z = lax.axis_index("z")
        other_x = 1 - my_x

        barrier_sem = pltpu.get_barrier_semaphore()
        pl.semaphore_signal(
            barrier_sem, inc=1,
            device_id=(other_x, my_y, my_z),
            device_id_type=pl.DeviceIdType.MESH,
        )
        pl.semaphore_wait(barrier_sem, 1)

        def exchange(keep_lo, send_lo):
            keep_cp = pltpu.make_async_copy(
                x_hbm.at[0, :, pl.ds(keep_lo, n)], res_ref, keep_sem
            )
            keep_cp.start()

            stage_cps = []
            for k in range(K):
                rows = pl.ds(CHUNK_OFF[k], CHUNK_ROWS[k])
                cp = pltpu.make_async_copy(
                    x_hbm.at[0, rows, pl.ds(send_lo, n)],
                    stage_ref.at[rows],
                    stage_sems.at[k],
                )
                cp.start()
                stage_cps.append(cp)

            rdmas = []
            for k in range(K):
                rows = pl.ds(CHUNK_OFF[k], CHUNK_ROWS[k])
                stage_cps[k].wait()
                send_bf16[rows, :] = stage_ref[rows, :].astype(jnp.bfloat16)
                rdma = pltpu.make_async_remote_copy(
                    src_ref=send_bf16.at[rows],
                    dst_ref=comm_bf16.at[rows],
                    send_sem=send_sems.at[k],
                    recv_sem=recv_sems.at[k],
                    device_id=(other_x, my_y, my_z),
                    device_id_type=pl.DeviceIdType.MESH,
                )
                rdma.start()
                rdmas.append(rdma)

            keep_cp.wait()
            out_cps = []
            for k in range(K):
                rows = pl.ds(CHUNK_OFF[k], CHUNK_ROWS[k])
                rdmas[k].wait_recv()
                res_ref[rows, :] = res_ref[rows, :] + comm_bf16[rows, :].astype(
                    jnp.float32
                )
                ocp = pltpu.make_async_copy(
                    res_ref.at[rows], out_hbm.at[rows], out_sems.at[k]
                )
                ocp.start()
                out_cps.append(ocp)
            for k in range(K):
                out_cps[k].wait()
                rdmas[k].wait_send()

        @pl.when(my_x == 0)
        def _():
            exchange(0, n)

        @pl.when(my_x == 1)
        def _():
            exchange(n, 0)

    return pl.pallas_call(
        body,
        out_shape=jax.ShapeDtypeStruct((m, n), x.dtype),
        in_specs=[pl.BlockSpec(memory_space=pl.ANY)],
        out_specs=pl.BlockSpec(memory_space=pl.ANY),
        scratch_shapes=[
            pltpu.VMEM((m, n), jnp.float32),
            pltpu.VMEM((m, n), jnp.float32),
            pltpu.VMEM((m, n), jnp.bfloat16),
            pltpu.VMEM((m, n), jnp.bfloat16),
            pltpu.SemaphoreType.DMA((K,)),
            pltpu.SemaphoreType.DMA((K,)),
            pltpu.SemaphoreType.DMA((K,)),
            pltpu.SemaphoreType.DMA((K,)),
            pltpu.SemaphoreType.DMA,
        ],
        compiler_params=pltpu.CompilerParams(
            collective_id=0,
            vmem_limit_bytes=60 * 1024 * 1024,
        ),
    )(x)


# device time: 108604 ns/iter; 1.0597x vs baseline; 1.0050x over previous
import jax
import jax.numpy as jnp
from jax import lax
from jax.experimental import pallas as pl
from jax.experimental.pallas import tpu as pltpu

CHUNK_ROWS = (128, 256, 512, 704, 704, 704, 704, 256, 128)
K = len(CHUNK_ROWS)
CHUNK_OFF = tuple(sum(CHUNK_ROWS[:i]) for i in range(K))


def kernel(x):
    _, m, n2 = x.shape
    n = n2 // 2
    assert sum(CHUNK_ROWS) == m

    def body(
        x_hbm,
        out_hbm,
        res_ref,
        stage_ref,
        send_bf16,
        comm_bf16,
        send_sems,
        recv_sems,
        stage_sems,
        out_sems,
        keep_sem,
    ):
        my_x = lax.axis_index("x")
        my_y = lax.axis_index("y")
        my_z = lax.axis_index("z")
        other_x = 1 - my_x

        barrier_sem = pltpu.get_barrier_semaphore()
        pl.semaphore_signal(
            barrier_sem, inc=1,
            device_id=(other_x, my_y, my_z),
            device_id_type=pl.DeviceIdType.MESH,
        )
        pl.semaphore_wait(barrier_sem, 1)

        def exchange(keep_lo, send_lo):
            keep_cp = pltpu.make_async_copy(
                x_hbm.at[0, :, pl.ds(keep_lo, n)], res_ref, keep_sem
            )
            keep_cp.start()

            stage_cps = []
            for k in range(K):
                rows = pl.ds(CHUNK_OFF[k], CHUNK_ROWS[k])
                cp = pltpu.make_async_copy(
                    x_hbm.at[0, rows, pl.ds(send_lo, n)],
                    stage_ref.at[rows],
                    stage_sems.at[k],
                )
                cp.start()
                stage_cps.append(cp)

            rdmas = []
            for k in range(K):
                rows = pl.ds(CHUNK_OFF[k], CHUNK_ROWS[k])
                stage_cps[k].wait()
                send_bf16[rows, :] = stage_ref[rows, :].astype(jnp.bfloat16)
                rdma = pltpu.make_async_remote_copy(
                    src_ref=send_bf16.at[rows],
                    dst_ref=comm_bf16.at[rows],
                    send_sem=send_sems.at[k],
                    recv_sem=recv_sems.at[k],
                    device_id=(other_x, my_y, my_z),
                    device_id_type=pl.DeviceIdType.MESH,
                )
                rdma.start()
                rdmas.append(rdma)

            keep_cp.wait()
            out_cps = []
            for k in range(K):
                rows = pl.ds(CHUNK_OFF[k], CHUNK_ROWS[k])
                rdmas[k].wait_recv()
                res_ref[rows, :] = res_ref[rows, :] + comm_bf16[rows, :].astype(
                    jnp.float32
                )
                ocp = pltpu.make_async_copy(
                    res_ref.at[rows], out_hbm.at[rows], out_sems.at[k]
                )
                ocp.start()
                out_cps.append(ocp)
            for k in range(K):
                out_cps[k].wait()
                rdmas[k].wait_send()

        @pl.when(my_x == 0)
        def _():
            exchange(0, n)

        @pl.when(my_x == 1)
        def _():
            exchange(n, 0)

    return pl.pallas_call(
        body,
        out_shape=jax.ShapeDtypeStruct((m, n), x.dtype),
        in_specs=[pl.BlockSpec(memory_space=pl.ANY)],
        out_specs=pl.BlockSpec(memory_space=pl.ANY),
        scratch_shapes=[
            pltpu.VMEM((m, n), jnp.float32),
            pltpu.VMEM((m, n), jnp.float32),
            pltpu.VMEM((m, n), jnp.bfloat16),
            pltpu.VMEM((m, n), jnp.bfloat16),
            pltpu.SemaphoreType.DMA((K,)),
            pltpu.SemaphoreType.DMA((K,)),
            pltpu.SemaphoreType.DMA((K,)),
            pltpu.SemaphoreType.DMA((K,)),
            pltpu.SemaphoreType.DMA,
        ],
        compiler_params=pltpu.CompilerParams(
            collective_id=0,
            vmem_limit_bytes=60 * 1024 * 1024,
        ),
    )(x)
